# Initial kernel scaffold; baseline (speedup 1.0000x reference)
import jax
import jax.numpy as jnp
from jax.experimental import pallas as pl


def kernel(x, edge_index, batch, W1, b1, W2, b2, W3, b3):
    raise NotImplementedError("write your pallas kernel here")


# speedup vs baseline: 6.6170x; 6.6170x over previous
import functools

import jax
import jax.numpy as jnp
from jax import lax
from jax.experimental import pallas as pl
from jax.experimental.pallas import tpu as pltpu
from jax.experimental.pallas import tpu_sc as plsc

NN = 10000
EE = 320000
DF = 128
NB = 64

NC, NS = 2, 16
NW = NC * NS
CHUNK = 128
CPW = 80
EPW = CPW * CHUNK
EPAD = NW * EPW
ACC_ROWS = 10240
ZROWS = ACC_ROWS // NS
OROWS = NN // NS
DW = 128

RB = 2000
GRID = NN // RB


def _deg_body(dstp, ones_h, zeros_h, out, dstv, onesv, acc):
    c = lax.axis_index("c")
    s = lax.axis_index("s")
    w = c * NS + s
    pltpu.sync_copy(zeros_h, acc.at[pl.ds(s * ZROWS, ZROWS)])
    pltpu.sync_copy(dstp.at[w], dstv)
    pltpu.sync_copy(ones_h, onesv)
    plsc.subcore_barrier()

    def step(j, carry):
        pltpu.sync_copy(onesv, acc.at[dstv.at[j]], add=True)
        return carry

    lax.fori_loop(0, CPW, step, 0)
    plsc.subcore_barrier()
    pltpu.sync_copy(acc.at[pl.ds(s * ZROWS, ZROWS)],
                    out.at[c].at[pl.ds(s * ZROWS, ZROWS)])


_deg_kernel = functools.partial(
    pl.kernel,
    out_type=jax.ShapeDtypeStruct((NC, ACC_ROWS, DW), jnp.float32),
    mesh=plsc.VectorSubcoreMesh(core_axis_name="c", subcore_axis_name="s"),
    scratch_types=[
        pltpu.VMEM((CPW, CHUNK), jnp.int32),
        pltpu.VMEM((CHUNK, DW), jnp.float32),
        pltpu.VMEM_SHARED((ACC_ROWS, DW), jnp.float32),
    ],
)(_deg_body)


def _agg_body(g_h, srcp, dstp, zeros_h, out, srcv, dstv, buf, sem, acc):
    c = lax.axis_index("c")
    s = lax.axis_index("s")
    w = c * NS + s
    pltpu.sync_copy(zeros_h, acc.at[pl.ds(s * ZROWS, ZROWS)])
    pltpu.sync_copy(srcp.at[w], srcv)
    pltpu.sync_copy(dstp.at[w], dstv)
    plsc.subcore_barrier()

    def step(j, carry):
        pltpu.async_copy(g_h.at[srcv.at[j]], buf, sem).wait()
        pltpu.sync_copy(buf, acc.at[dstv.at[j]], add=True)
        return carry

    lax.fori_loop(0, CPW, step, 0)
    plsc.subcore_barrier()
    pltpu.sync_copy(acc.at[pl.ds(s * ZROWS, ZROWS)],
                    out.at[c].at[pl.ds(s * ZROWS, ZROWS)])


_agg_kernel = functools.partial(
    pl.kernel,
    out_type=jax.ShapeDtypeStruct((NC, ACC_ROWS, DF), jnp.float32),
    mesh=plsc.VectorSubcoreMesh(core_axis_name="c", subcore_axis_name="s"),
    scratch_types=[
        pltpu.VMEM((CPW, CHUNK), jnp.int32),
        pltpu.VMEM((CPW, CHUNK), jnp.int32),
        pltpu.VMEM((CHUNK, DF), jnp.float32),
        pltpu.SemaphoreType.DMA,
        pltpu.VMEM_SHARED((ACC_ROWS, DF), jnp.float32),
    ],
)(_agg_body)


def _tc0_body(x_ref, w_ref, pd_ref, g_ref, dinv_ref):
    deg = 1.0 + pd_ref[0, :, 0:1] + pd_ref[1, :, 0:1]
    dinvb = jnp.broadcast_to(lax.rsqrt(deg), (RB, DF))
    h = jnp.dot(x_ref[...], w_ref[...], preferred_element_type=jnp.float32)
    g_ref[...] = h * dinvb
    dinv_ref[...] = dinvb


def _tc0(x, w1, pd):
    return pl.pallas_call(
        _tc0_body,
        grid=(GRID,),
        in_specs=[
            pl.BlockSpec((RB, DF), lambda i: (i, 0)),
            pl.BlockSpec((DF, DF), lambda i: (0, 0)),
            pl.BlockSpec((NC, RB, DW), lambda i: (0, i, 0)),
        ],
        out_specs=[
            pl.BlockSpec((RB, DF), lambda i: (i, 0)),
            pl.BlockSpec((RB, DF), lambda i: (i, 0)),
        ],
        out_shape=[
            jax.ShapeDtypeStruct((NN, DF), jnp.float32),
            jax.ShapeDtypeStruct((NN, DF), jnp.float32),
        ],
    )(x, w1, pd)


def _tcmid_body(p_ref, g_ref, dinv_ref, b_ref, w_ref, out_ref):
    dinvb = dinv_ref[...]
    xr = jax.nn.relu(dinvb * (p_ref[0] + p_ref[1] + g_ref[...])
                     + b_ref[...])
    out_ref[...] = jnp.dot(xr, w_ref[...],
                           preferred_element_type=jnp.float32) * dinvb


def _tcmid(p, g, dinvb, b, w):
    return pl.pallas_call(
        _tcmid_body,
        grid=(GRID,),
        in_specs=[
            pl.BlockSpec((NC, RB, DF), lambda i: (0, i, 0)),
            pl.BlockSpec((RB, DF), lambda i: (i, 0)),
            pl.BlockSpec((RB, DF), lambda i: (i, 0)),
            pl.BlockSpec((1, DF), lambda i: (0, 0)),
            pl.BlockSpec((DF, DF), lambda i: (0, 0)),
        ],
        out_specs=pl.BlockSpec((RB, DF), lambda i: (i, 0)),
        out_shape=jax.ShapeDtypeStruct((NN, DF), jnp.float32),
    )(p, g, dinvb, b, w)


def _tclast_body(p_ref, g_ref, dinv_ref, b_ref, binfo_ref, out_ref):
    i = pl.program_id(0)
    x4 = jax.nn.relu(dinv_ref[...] * (p_ref[0] + p_ref[1] + g_ref[...])
                     + b_ref[...])
    lo = binfo_ref[0:1, :]
    up = binfo_ref[1:2, :]
    recip = 1.0 / jnp.maximum(up - lo, 1.0)
    rid = (lax.broadcasted_iota(jnp.int32, (RB, NB), 0)
           + i * RB).astype(jnp.float32)
    mt = jnp.where((rid >= lo) & (rid < up), recip,
                   jnp.zeros((RB, NB), jnp.float32))
    contrib = lax.dot_general(mt, x4, (((0,), (0,)), ((), ())),
                              preferred_element_type=jnp.float32)

    @pl.when(i == 0)
    def _():
        out_ref[...] = contrib

    @pl.when(i > 0)
    def _():
        out_ref[...] += contrib


def _tclast(p, g, dinvb, b, binfo):
    return pl.pallas_call(
        _tclast_body,
        grid=(GRID,),
        in_specs=[
            pl.BlockSpec((NC, RB, DF), lambda i: (0, i, 0)),
            pl.BlockSpec((RB, DF), lambda i: (i, 0)),
            pl.BlockSpec((RB, DF), lambda i: (i, 0)),
            pl.BlockSpec((1, DF), lambda i: (0, 0)),
            pl.BlockSpec((8, NB), lambda i: (0, 0)),
        ],
        out_specs=pl.BlockSpec((NB, DF), lambda i: (0, 0)),
        out_shape=jax.ShapeDtypeStruct((NB, DF), jnp.float32),
        compiler_params=pltpu.CompilerParams(
            dimension_semantics=("arbitrary",)),
    )(p, g, dinvb, b, binfo)


def kernel(x, edge_index, batch, W1, b1, W2, b2, W3, b3):
    src = edge_index[0]
    dst = edge_index[1]
    pad = EPAD - EE
    srcp = jnp.concatenate(
        [src, jnp.zeros((pad,), jnp.int32)]).reshape(NW, CPW, CHUNK)
    dstp = jnp.concatenate(
        [dst, jnp.full((pad,), NN, jnp.int32)]).reshape(NW, CPW, CHUNK)

    zeros_f = jnp.zeros((ZROWS, DF), jnp.float32)
    zeros_d = jnp.zeros((ZROWS, DW), jnp.float32)
    ones_d = jnp.ones((CHUNK, DW), jnp.float32)

    lo = batch[:NB].astype(jnp.float32)
    up = batch[1:].astype(jnp.float32)
    binfo = jnp.concatenate(
        [lo[None], up[None], jnp.zeros((6, NB), jnp.float32)], axis=0)

    b1r = b1.reshape(1, DF)
    b2r = b2.reshape(1, DF)
    b3r = b3.reshape(1, DF)

    pd = _deg_kernel(dstp, ones_d, zeros_d)
    g1, dinvb = _tc0(x, W1, pd)
    p = _agg_kernel(g1, srcp, dstp, zeros_f)
    g2 = _tcmid(p, g1, dinvb, b1r, W2)
    p = _agg_kernel(g2, srcp, dstp, zeros_f)
    g3 = _tcmid(p, g2, dinvb, b2r, W3)
    p = _agg_kernel(g3, srcp, dstp, zeros_f)
    return _tclast(p, g3, dinvb, b3r, binfo)


# speedup vs baseline: 7.2759x; 1.0996x over previous
import functools

import jax
import jax.numpy as jnp
from jax import lax
from jax.experimental import pallas as pl
from jax.experimental.pallas import tpu as pltpu
from jax.experimental.pallas import tpu_sc as plsc

NN = 10000
EE = 320000
DF = 128
NB = 64

NC, NS = 2, 16
NW = NC * NS
CHUNK = 128
CPW = 80
WIN = 40
EPW = CPW * CHUNK
EPAD = NW * EPW
ACC_ROWS = 10240
ZROWS = ACC_ROWS // NS
OROWS = NN // NS
DW = 128

RB = 2000
GRID = NN // RB


def _deg_body(dstp, ones_h, zeros_h, out, dstv, onesv, acc):
    c = lax.axis_index("c")
    s = lax.axis_index("s")
    w = c * NS + s
    pltpu.sync_copy(zeros_h, acc.at[pl.ds(s * ZROWS, ZROWS)])
    pltpu.sync_copy(dstp.at[w], dstv)
    pltpu.sync_copy(ones_h, onesv)
    plsc.subcore_barrier()

    def step(j, carry):
        pltpu.sync_copy(onesv, acc.at[dstv.at[j]], add=True)
        return carry

    lax.fori_loop(0, CPW, step, 0)
    plsc.subcore_barrier()
    pltpu.sync_copy(acc.at[pl.ds(s * ZROWS, ZROWS)],
                    out.at[c].at[pl.ds(s * ZROWS, ZROWS)])


_deg_kernel = functools.partial(
    pl.kernel,
    out_type=jax.ShapeDtypeStruct((NC, ACC_ROWS, DW), jnp.float32),
    mesh=plsc.VectorSubcoreMesh(core_axis_name="c", subcore_axis_name="s"),
    scratch_types=[
        pltpu.VMEM((CPW, CHUNK), jnp.int32),
        pltpu.VMEM((CHUNK, DW), jnp.float32),
        pltpu.VMEM_SHARED((ACC_ROWS, DW), jnp.float32),
    ],
)(_deg_body)


def _agg_body(g_h, srcp, dstp, zeros_h, out,
              srcv, dstv, buf0, buf1, sem0, sem1, acc):
    c = lax.axis_index("c")
    s = lax.axis_index("s")
    w = c * NS + s
    pltpu.sync_copy(zeros_h, acc.at[pl.ds(s * ZROWS, ZROWS)])
    plsc.subcore_barrier()

    for h in range(CPW // WIN):
        pltpu.sync_copy(srcp.at[w].at[pl.ds(h * WIN, WIN)], srcv)
        pltpu.sync_copy(dstp.at[w].at[pl.ds(h * WIN, WIN)], dstv)
        pltpu.async_copy(g_h.at[srcv.at[0]], buf0, sem0)
        pltpu.async_copy(g_h.at[srcv.at[1]], buf1, sem1)

        def step(j, carry):
            pltpu.make_async_copy(g_h.at[srcv.at[2 * j]], buf0, sem0).wait()
            pltpu.sync_copy(buf0, acc.at[dstv.at[2 * j]], add=True)
            pltpu.async_copy(g_h.at[srcv.at[2 * j + 2]], buf0, sem0)
            pltpu.make_async_copy(
                g_h.at[srcv.at[2 * j + 1]], buf1, sem1).wait()
            pltpu.sync_copy(buf1, acc.at[dstv.at[2 * j + 1]], add=True)
            pltpu.async_copy(g_h.at[srcv.at[2 * j + 3]], buf1, sem1)
            return carry

        lax.fori_loop(0, WIN // 2 - 1, step, 0)
        pltpu.make_async_copy(g_h.at[srcv.at[WIN - 2]], buf0, sem0).wait()
        pltpu.sync_copy(buf0, acc.at[dstv.at[WIN - 2]], add=True)
        pltpu.make_async_copy(g_h.at[srcv.at[WIN - 1]], buf1, sem1).wait()
        pltpu.sync_copy(buf1, acc.at[dstv.at[WIN - 1]], add=True)
    plsc.subcore_barrier()
    pltpu.sync_copy(acc.at[pl.ds(s * ZROWS, ZROWS)],
                    out.at[c].at[pl.ds(s * ZROWS, ZROWS)])


_agg_kernel = functools.partial(
    pl.kernel,
    out_type=jax.ShapeDtypeStruct((NC, ACC_ROWS, DF), jnp.float32),
    mesh=plsc.VectorSubcoreMesh(core_axis_name="c", subcore_axis_name="s"),
    scratch_types=[
        pltpu.VMEM((WIN, CHUNK), jnp.int32),
        pltpu.VMEM((WIN, CHUNK), jnp.int32),
        pltpu.VMEM((CHUNK, DF), jnp.float32),
        pltpu.VMEM((CHUNK, DF), jnp.float32),
        pltpu.SemaphoreType.DMA,
        pltpu.SemaphoreType.DMA,
        pltpu.VMEM_SHARED((ACC_ROWS, DF), jnp.float32),
    ],
)(_agg_body)


def _tc0_body(x_ref, w_ref, pd_ref, g_ref, dinv_ref):
    deg = 1.0 + pd_ref[0, :, 0:1] + pd_ref[1, :, 0:1]
    dinvb = jnp.broadcast_to(lax.rsqrt(deg), (RB, DF))
    h = jnp.dot(x_ref[...], w_ref[...], preferred_element_type=jnp.float32)
    g_ref[...] = h * dinvb
    dinv_ref[...] = dinvb


def _tc0(x, w1, pd):
    return pl.pallas_call(
        _tc0_body,
        grid=(GRID,),
        in_specs=[
            pl.BlockSpec((RB, DF), lambda i: (i, 0)),
            pl.BlockSpec((DF, DF), lambda i: (0, 0)),
            pl.BlockSpec((NC, RB, DW), lambda i: (0, i, 0)),
        ],
        out_specs=[
            pl.BlockSpec((RB, DF), lambda i: (i, 0)),
            pl.BlockSpec((RB, DF), lambda i: (i, 0)),
        ],
        out_shape=[
            jax.ShapeDtypeStruct((NN, DF), jnp.float32),
            jax.ShapeDtypeStruct((NN, DF), jnp.float32),
        ],
    )(x, w1, pd)


def _tcmid_body(p_ref, g_ref, dinv_ref, b_ref, w_ref, out_ref):
    dinvb = dinv_ref[...]
    xr = jax.nn.relu(dinvb * (p_ref[0] + p_ref[1] + g_ref[...])
                     + b_ref[...])
    out_ref[...] = jnp.dot(xr, w_ref[...],
                           preferred_element_type=jnp.float32) * dinvb


def _tcmid(p, g, dinvb, b, w):
    return pl.pallas_call(
        _tcmid_body,
        grid=(GRID,),
        in_specs=[
            pl.BlockSpec((NC, RB, DF), lambda i: (0, i, 0)),
            pl.BlockSpec((RB, DF), lambda i: (i, 0)),
            pl.BlockSpec((RB, DF), lambda i: (i, 0)),
            pl.BlockSpec((1, DF), lambda i: (0, 0)),
            pl.BlockSpec((DF, DF), lambda i: (0, 0)),
        ],
        out_specs=pl.BlockSpec((RB, DF), lambda i: (i, 0)),
        out_shape=jax.ShapeDtypeStruct((NN, DF), jnp.float32),
    )(p, g, dinvb, b, w)


def _tclast_body(p_ref, g_ref, dinv_ref, b_ref, binfo_ref, out_ref):
    i = pl.program_id(0)
    x4 = jax.nn.relu(dinv_ref[...] * (p_ref[0] + p_ref[1] + g_ref[...])
                     + b_ref[...])
    lo = binfo_ref[0:1, :]
    up = binfo_ref[1:2, :]
    recip = 1.0 / jnp.maximum(up - lo, 1.0)
    rid = (lax.broadcasted_iota(jnp.int32, (RB, NB), 0)
           + i * RB).astype(jnp.float32)
    mt = jnp.where((rid >= lo) & (rid < up), recip,
                   jnp.zeros((RB, NB), jnp.float32))
    contrib = lax.dot_general(mt, x4, (((0,), (0,)), ((), ())),
                              preferred_element_type=jnp.float32)

    @pl.when(i == 0)
    def _():
        out_ref[...] = contrib

    @pl.when(i > 0)
    def _():
        out_ref[...] += contrib


def _tclast(p, g, dinvb, b, binfo):
    return pl.pallas_call(
        _tclast_body,
        grid=(GRID,),
        in_specs=[
            pl.BlockSpec((NC, RB, DF), lambda i: (0, i, 0)),
            pl.BlockSpec((RB, DF), lambda i: (i, 0)),
            pl.BlockSpec((RB, DF), lambda i: (i, 0)),
            pl.BlockSpec((1, DF), lambda i: (0, 0)),
            pl.BlockSpec((8, NB), lambda i: (0, 0)),
        ],
        out_specs=pl.BlockSpec((NB, DF), lambda i: (0, 0)),
        out_shape=jax.ShapeDtypeStruct((NB, DF), jnp.float32),
        compiler_params=pltpu.CompilerParams(
            dimension_semantics=("arbitrary",)),
    )(p, g, dinvb, b, binfo)


def kernel(x, edge_index, batch, W1, b1, W2, b2, W3, b3):
    src = edge_index[0]
    dst = edge_index[1]
    pad = EPAD - EE
    srcp = jnp.concatenate(
        [src, jnp.zeros((pad,), jnp.int32)]).reshape(NW, CPW, CHUNK)
    dstp = jnp.concatenate(
        [dst, jnp.full((pad,), NN, jnp.int32)]).reshape(NW, CPW, CHUNK)

    zeros_f = jnp.zeros((ZROWS, DF), jnp.float32)
    zeros_d = jnp.zeros((ZROWS, DW), jnp.float32)
    ones_d = jnp.ones((CHUNK, DW), jnp.float32)

    lo = batch[:NB].astype(jnp.float32)
    up = batch[1:].astype(jnp.float32)
    binfo = jnp.concatenate(
        [lo[None], up[None], jnp.zeros((6, NB), jnp.float32)], axis=0)

    b1r = b1.reshape(1, DF)
    b2r = b2.reshape(1, DF)
    b3r = b3.reshape(1, DF)

    pd = _deg_kernel(dstp, ones_d, zeros_d)
    g1, dinvb = _tc0(x, W1, pd)
    p = _agg_kernel(g1, srcp, dstp, zeros_f)
    g2 = _tcmid(p, g1, dinvb, b1r, W2)
    p = _agg_kernel(g2, srcp, dstp, zeros_f)
    g3 = _tcmid(p, g2, dinvb, b2r, W3)
    p = _agg_kernel(g3, srcp, dstp, zeros_f)
    return _tclast(p, g3, dinvb, b3r, binfo)


# speedup vs baseline: 7.6868x; 1.0565x over previous
import functools

import jax
import jax.numpy as jnp
from jax import lax
from jax.experimental import pallas as pl
from jax.experimental.pallas import tpu as pltpu
from jax.experimental.pallas import tpu_sc as plsc

NN = 10000
EE = 320000
DF = 128
NB = 64

NC, NS = 2, 16
NW = NC * NS
CHUNK = 128
CPW = 80
WIN = 40
EPW = CPW * CHUNK
EPAD = NW * EPW
ACC_ROWS = 10112
ZROWS = ACC_ROWS // NS
OROWS = NN // NS
DW = 128

RB = 2000
GRID = NN // RB


def _deg_body(dstp, ones_h, zeros_h, out, dstv, onesv, acc):
    c = lax.axis_index("c")
    s = lax.axis_index("s")
    w = c * NS + s
    pltpu.sync_copy(zeros_h, acc.at[pl.ds(s * ZROWS, ZROWS)])
    pltpu.sync_copy(dstp.at[w], dstv)
    pltpu.sync_copy(ones_h, onesv)
    plsc.subcore_barrier()

    def step(j, carry):
        pltpu.sync_copy(onesv, acc.at[dstv.at[j]], add=True)
        return carry

    lax.fori_loop(0, CPW, step, 0)
    plsc.subcore_barrier()
    pltpu.sync_copy(acc.at[pl.ds(s * ZROWS, ZROWS)],
                    out.at[c].at[pl.ds(s * ZROWS, ZROWS)])


_deg_kernel = functools.partial(
    pl.kernel,
    out_type=jax.ShapeDtypeStruct((NC, ACC_ROWS, DW), jnp.float32),
    mesh=plsc.VectorSubcoreMesh(core_axis_name="c", subcore_axis_name="s"),
    scratch_types=[
        pltpu.VMEM((CPW, CHUNK), jnp.int32),
        pltpu.VMEM((CHUNK, DW), jnp.float32),
        pltpu.VMEM_SHARED((ACC_ROWS, DW), jnp.float32),
    ],
)(_deg_body)


def _agg_body(g_h, srcp, dstp, zeros_h, out,
              srcv, dstv, buf0, buf1, gs0, gs1, ss0, ss1, acc):
    c = lax.axis_index("c")
    s = lax.axis_index("s")
    w = c * NS + s
    pltpu.sync_copy(zeros_h, acc.at[pl.ds(s * ZROWS, ZROWS)])
    plsc.subcore_barrier()

    for h in range(CPW // WIN):
        pltpu.sync_copy(srcp.at[w].at[pl.ds(h * WIN, WIN)], srcv)
        pltpu.sync_copy(dstp.at[w].at[pl.ds(h * WIN, WIN)], dstv)
        pltpu.async_copy(g_h.at[srcv.at[0]], buf0, gs0)
        pltpu.async_copy(g_h.at[srcv.at[1]], buf1, gs1)

        def step(j, carry):
            pltpu.make_async_copy(g_h.at[srcv.at[2 * j]], buf0, gs0).wait()
            pltpu.async_copy(buf0, acc.at[dstv.at[2 * j]], ss0, add=True)
            pltpu.make_async_copy(
                g_h.at[srcv.at[2 * j + 1]], buf1, gs1).wait()
            pltpu.async_copy(buf1, acc.at[dstv.at[2 * j + 1]], ss1, add=True)
            pltpu.make_async_copy(g_h.at[srcv.at[2 * j]], buf0, ss0).wait()
            pltpu.async_copy(g_h.at[srcv.at[2 * j + 2]], buf0, gs0)
            pltpu.make_async_copy(g_h.at[srcv.at[2 * j + 1]], buf1, ss1).wait()
            pltpu.async_copy(g_h.at[srcv.at[2 * j + 3]], buf1, gs1)
            return carry

        lax.fori_loop(0, WIN // 2 - 1, step, 0)
        pltpu.make_async_copy(g_h.at[srcv.at[WIN - 2]], buf0, gs0).wait()
        pltpu.async_copy(buf0, acc.at[dstv.at[WIN - 2]], ss0, add=True)
        pltpu.make_async_copy(g_h.at[srcv.at[WIN - 1]], buf1, gs1).wait()
        pltpu.async_copy(buf1, acc.at[dstv.at[WIN - 1]], ss1, add=True)
        pltpu.make_async_copy(g_h.at[srcv.at[WIN - 2]], buf0, ss0).wait()
        pltpu.make_async_copy(g_h.at[srcv.at[WIN - 1]], buf1, ss1).wait()
    plsc.subcore_barrier()
    pltpu.sync_copy(acc.at[pl.ds(s * ZROWS, ZROWS)],
                    out.at[c].at[pl.ds(s * ZROWS, ZROWS)])


_agg_kernel = functools.partial(
    pl.kernel,
    out_type=jax.ShapeDtypeStruct((NC, ACC_ROWS, DF), jnp.float32),
    mesh=plsc.VectorSubcoreMesh(core_axis_name="c", subcore_axis_name="s"),
    scratch_types=[
        pltpu.VMEM((WIN, CHUNK), jnp.int32),
        pltpu.VMEM((WIN, CHUNK), jnp.int32),
        pltpu.VMEM((CHUNK, DF), jnp.float32),
        pltpu.VMEM((CHUNK, DF), jnp.float32),
        pltpu.SemaphoreType.DMA,
        pltpu.SemaphoreType.DMA,
        pltpu.SemaphoreType.DMA,
        pltpu.SemaphoreType.DMA,
        pltpu.VMEM_SHARED((ACC_ROWS, DF), jnp.float32),
    ],
)(_agg_body)


def _tc0_body(x_ref, w_ref, pd_ref, g_ref, dinv_ref):
    deg = 1.0 + pd_ref[0, :, 0:1] + pd_ref[1, :, 0:1]
    dinvb = jnp.broadcast_to(lax.rsqrt(deg), (RB, DF))
    h = jnp.dot(x_ref[...], w_ref[...], preferred_element_type=jnp.float32)
    g_ref[...] = h * dinvb
    dinv_ref[...] = dinvb


def _tc0(x, w1, pd):
    return pl.pallas_call(
        _tc0_body,
        grid=(GRID,),
        in_specs=[
            pl.BlockSpec((RB, DF), lambda i: (i, 0)),
            pl.BlockSpec((DF, DF), lambda i: (0, 0)),
            pl.BlockSpec((NC, RB, DW), lambda i: (0, i, 0)),
        ],
        out_specs=[
            pl.BlockSpec((RB, DF), lambda i: (i, 0)),
            pl.BlockSpec((RB, DF), lambda i: (i, 0)),
        ],
        out_shape=[
            jax.ShapeDtypeStruct((NN, DF), jnp.float32),
            jax.ShapeDtypeStruct((NN, DF), jnp.float32),
        ],
    )(x, w1, pd)


def _tcmid_body(p_ref, g_ref, dinv_ref, b_ref, w_ref, out_ref):
    dinvb = dinv_ref[...]
    xr = jax.nn.relu(dinvb * (p_ref[0] + p_ref[1] + g_ref[...])
                     + b_ref[...])
    out_ref[...] = jnp.dot(xr, w_ref[...],
                           preferred_element_type=jnp.float32) * dinvb


def _tcmid(p, g, dinvb, b, w):
    return pl.pallas_call(
        _tcmid_body,
        grid=(GRID,),
        in_specs=[
            pl.BlockSpec((NC, RB, DF), lambda i: (0, i, 0)),
            pl.BlockSpec((RB, DF), lambda i: (i, 0)),
            pl.BlockSpec((RB, DF), lambda i: (i, 0)),
            pl.BlockSpec((1, DF), lambda i: (0, 0)),
            pl.BlockSpec((DF, DF), lambda i: (0, 0)),
        ],
        out_specs=pl.BlockSpec((RB, DF), lambda i: (i, 0)),
        out_shape=jax.ShapeDtypeStruct((NN, DF), jnp.float32),
    )(p, g, dinvb, b, w)


def _tclast_body(p_ref, g_ref, dinv_ref, b_ref, binfo_ref, out_ref):
    i = pl.program_id(0)
    x4 = jax.nn.relu(dinv_ref[...] * (p_ref[0] + p_ref[1] + g_ref[...])
                     + b_ref[...])
    lo = binfo_ref[0:1, :]
    up = binfo_ref[1:2, :]
    recip = 1.0 / jnp.maximum(up - lo, 1.0)
    rid = (lax.broadcasted_iota(jnp.int32, (RB, NB), 0)
           + i * RB).astype(jnp.float32)
    mt = jnp.where((rid >= lo) & (rid < up), recip,
                   jnp.zeros((RB, NB), jnp.float32))
    contrib = lax.dot_general(mt, x4, (((0,), (0,)), ((), ())),
                              preferred_element_type=jnp.float32)

    @pl.when(i == 0)
    def _():
        out_ref[...] = contrib

    @pl.when(i > 0)
    def _():
        out_ref[...] += contrib


def _tclast(p, g, dinvb, b, binfo):
    return pl.pallas_call(
        _tclast_body,
        grid=(GRID,),
        in_specs=[
            pl.BlockSpec((NC, RB, DF), lambda i: (0, i, 0)),
            pl.BlockSpec((RB, DF), lambda i: (i, 0)),
            pl.BlockSpec((RB, DF), lambda i: (i, 0)),
            pl.BlockSpec((1, DF), lambda i: (0, 0)),
            pl.BlockSpec((8, NB), lambda i: (0, 0)),
        ],
        out_specs=pl.BlockSpec((NB, DF), lambda i: (0, 0)),
        out_shape=jax.ShapeDtypeStruct((NB, DF), jnp.float32),
        compiler_params=pltpu.CompilerParams(
            dimension_semantics=("arbitrary",)),
    )(p, g, dinvb, b, binfo)


def kernel(x, edge_index, batch, W1, b1, W2, b2, W3, b3):
    src = edge_index[0]
    dst = edge_index[1]
    pad = EPAD - EE
    srcp = jnp.concatenate(
        [src, jnp.zeros((pad,), jnp.int32)]).reshape(NW, CPW, CHUNK)
    dstp = jnp.concatenate(
        [dst, jnp.full((pad,), NN, jnp.int32)]).reshape(NW, CPW, CHUNK)

    zeros_f = jnp.zeros((ZROWS, DF), jnp.float32)
    zeros_d = jnp.zeros((ZROWS, DW), jnp.float32)
    ones_d = jnp.ones((CHUNK, DW), jnp.float32)

    lo = batch[:NB].astype(jnp.float32)
    up = batch[1:].astype(jnp.float32)
    binfo = jnp.concatenate(
        [lo[None], up[None], jnp.zeros((6, NB), jnp.float32)], axis=0)

    b1r = b1.reshape(1, DF)
    b2r = b2.reshape(1, DF)
    b3r = b3.reshape(1, DF)

    pd = _deg_kernel(dstp, ones_d, zeros_d)
    g1, dinvb = _tc0(x, W1, pd)
    p = _agg_kernel(g1, srcp, dstp, zeros_f)
    g2 = _tcmid(p, g1, dinvb, b1r, W2)
    p = _agg_kernel(g2, srcp, dstp, zeros_f)
    g3 = _tcmid(p, g2, dinvb, b2r, W3)
    p = _agg_kernel(g3, srcp, dstp, zeros_f)
    return _tclast(p, g3, dinvb, b3r, binfo)
